# baseline (speedup 1.0000x reference)
import functools
import jax
import jax.numpy as jnp
from jax import lax
from jax.experimental import pallas as pl
from jax.experimental.pallas import tpu as pltpu
from jax.experimental.pallas import tpu_sc as plsc

T = 2048
C = 768
C2 = 2 * C
K = 5
NL = 3
TPAD = T + 8
NT = 4
TB = T // NT
CODE = 32
NCODES = 8192
KB = 1024
NKB = NCODES // KB


def _gate(xin):
    return jnp.tanh(xin[:, :C]) * (
        0.5 * (1.0 + jnp.tanh(xin[:, C:] * 0.5)))


def _wn_body(x0_ref, w_ref, b_ref, rsw_ref, rsb_ref, rslw_ref, rslb_ref,
             o_ref, xpad_s, xin_s, acc_s, xres_s):
        l = pl.program_id(0)
        k = pl.program_id(1)

        @pl.when((l == 0) & (k == 0))
        def _():
            xpad_s[...] = jnp.zeros((TPAD, C), jnp.bfloat16)
            xpad_s[2:2 + T, :] = x0_ref[...].astype(jnp.bfloat16)
            xres_s[...] = x0_ref[...]
            acc_s[...] = jnp.zeros((T, C), jnp.float32)

        w = w_ref[0, 0]
        for i in range(K):
            @pl.when(k == i)
            def _(i=i):
                contrib = lax.dot_general(
                    xpad_s[i:i + T, :], w, (((1,), (1,)), ((), ())),
                    preferred_element_type=jnp.float32)
                if i == 0:
                    xin_s[...] = b_ref[0] + contrib
                else:
                    xin_s[...] += contrib

        @pl.when(k == K - 1)
        def _():
            @pl.when(l < NL - 1)
            def _():
                rsw = rsw_ref[0]
                rsb = rsb_ref[0]
                for ci in range(NT):
                    r0 = ci * TB
                    a = _gate(xin_s[r0:r0 + TB, :]).astype(jnp.bfloat16)
                    rsa = lax.dot_general(
                        a, rsw, (((1,), (1,)), ((), ())),
                        preferred_element_type=jnp.float32) + rsb
                    xn = xres_s[r0:r0 + TB, :] + rsa[:, :C]
                    xres_s[r0:r0 + TB, :] = xn
                    xpad_s[2 + r0:2 + r0 + TB, :] = xn.astype(jnp.bfloat16)
                    acc_s[r0:r0 + TB, :] += rsa[:, C:]

            @pl.when(l == NL - 1)
            def _():
                rslw = rslw_ref[...]
                rslb = rslb_ref[...]
                for ci in range(NT):
                    r0 = ci * TB
                    a = _gate(xin_s[r0:r0 + TB, :]).astype(jnp.bfloat16)
                    rsa = lax.dot_general(
                        a, rslw, (((1,), (1,)), ((), ())),
                        preferred_element_type=jnp.float32) + rslb
                    out = acc_s[r0:r0 + TB, :] + rsa
                    o_ref[r0:r0 + TB, :] = out


def _wn_stack(x0, W_all, b_all, rs_W, rs_b, rslw, rslb):
    in_specs = [
        pl.BlockSpec((T, C), lambda l, k: (0, 0)),
        pl.BlockSpec((1, 1, C2, C), lambda l, k: (l, k, 0, 0)),
        pl.BlockSpec((1, 1, C2), lambda l, k: (l, 0, 0)),
        pl.BlockSpec((1, C2, C), lambda l, k: (jnp.minimum(l, NL - 2), 0, 0)),
        pl.BlockSpec((1, 1, C2), lambda l, k: (jnp.minimum(l, NL - 2), 0, 0)),
        pl.BlockSpec((C, C), lambda l, k: (0, 0)),
        pl.BlockSpec((1, C), lambda l, k: (0, 0)),
    ]
    args = [x0, W_all, b_all, rs_W, rs_b, rslw, rslb]
    out_spec = pl.BlockSpec((T, C), lambda l, k: (0, 0))
    out_shape = jax.ShapeDtypeStruct((T, C), jnp.float32)
    return pl.pallas_call(
        _wn_body,
        grid=(NL, K),
        in_specs=in_specs,
        out_specs=out_spec,
        out_shape=out_shape,
        scratch_shapes=[
            pltpu.VMEM((TPAD, C), jnp.bfloat16),
            pltpu.VMEM((T, C2), jnp.float32),
            pltpu.VMEM((T, C), jnp.float32),
            pltpu.VMEM((T, C), jnp.float32),
        ],
        compiler_params=pltpu.CompilerParams(
            dimension_semantics=("arbitrary", "arbitrary")),
    )(*args)


def _dist_body(x_ref, w_ref, b_ref, e_ref, idx_ref, xp_ref, bv_ref, xn_s):
    j = pl.program_id(0)

    @pl.when(j == 0)
    def _():
        xp = lax.dot_general(x_ref[...], w_ref[...], (((1,), (1,)), ((), ())),
                             preferred_element_type=jnp.float32) + b_ref[...]
        nn = jnp.sqrt(jnp.sum(xp * xp, axis=1, keepdims=True))
        xp_ref[...] = xp
        xn_s[...] = xp / jnp.clip(nn, 1e-12, None)

    e = e_ref[...]
    n = jnp.sqrt(jnp.sum(e * e, axis=1, keepdims=True))
    en = e / jnp.clip(n, 1e-12, None)
    d = lax.dot_general(xn_s[...], en, (((1,), (1,)), ((), ())),
                        preferred_element_type=jnp.float32)
    bv = jnp.max(d, axis=1, keepdims=True)
    bi = jnp.argmax(d, axis=1).astype(jnp.int32)[:, None] + j * KB

    @pl.when(j == 0)
    def _():
        bv_ref[...] = bv
        idx_ref[...] = bi

    @pl.when(j > 0)
    def _():
        better = bv > bv_ref[...]
        idx_ref[...] = jnp.where(better, bi, idx_ref[...])
        bv_ref[...] = jnp.where(better, bv, bv_ref[...])


def _vq_argmax(x_enc, vqw, vqb, embed):
    return pl.pallas_call(
        _dist_body,
        grid=(NKB,),
        in_specs=[
            pl.BlockSpec((T, C), lambda j: (0, 0)),
            pl.BlockSpec((CODE, C), lambda j: (0, 0)),
            pl.BlockSpec((1, CODE), lambda j: (0, 0)),
            pl.BlockSpec((KB, CODE), lambda j: (j, 0)),
        ],
        out_specs=[pl.BlockSpec((T, 1), lambda j: (0, 0)),
                   pl.BlockSpec((T, CODE), lambda j: (0, 0))],
        out_shape=[jax.ShapeDtypeStruct((T, 1), jnp.int32),
                   jax.ShapeDtypeStruct((T, CODE), jnp.float32)],
        scratch_shapes=[pltpu.VMEM((T, 1), jnp.float32),
                        pltpu.VMEM((T, CODE), jnp.float32)],
    )(x_enc, vqw, vqb.reshape(1, CODE), embed)


_NC = 2
_NS = 16
_NW = _NC * _NS
_BPW = T // _NW


def _sc_gather_rows(table, idx):
    mesh = plsc.VectorSubcoreMesh(core_axis_name="c", subcore_axis_name="s")

    @functools.partial(
        pl.kernel, mesh=mesh,
        compiler_params=pltpu.CompilerParams(use_tc_tiling_on_sc=False),
        out_type=jax.ShapeDtypeStruct((T, CODE), jnp.float32),
        scratch_types=[
            pltpu.VMEM((_BPW,), jnp.int32),
            pltpu.VMEM((_BPW, CODE), jnp.float32),
            pltpu.SemaphoreType.DMA,
        ],
    )
    def gather_k(table_hbm, idx_hbm, out_hbm, idx_v, rows_v, sem):
        wid = lax.axis_index("s") * _NC + lax.axis_index("c")
        base = wid * _BPW
        pltpu.sync_copy(idx_hbm.at[pl.ds(base, _BPW)], idx_v)
        pltpu.async_copy(table_hbm.at[idx_v], rows_v, sem).wait()
        pltpu.sync_copy(rows_v, out_hbm.at[pl.ds(base, _BPW)])

    return gather_k(table, idx)


def _commit_proj_body(raw_ref, xp_ref, w_ref, b_ref, q_ref, cm_ref):
    raw = raw_ref[...]
    n = jnp.sqrt(jnp.sum(raw * raw, axis=1, keepdims=True))
    qn = raw / jnp.clip(n, 1e-12, None)
    diff = qn - xp_ref[...]
    cm_ref[...] = jnp.sum(diff * diff)[None, None]
    q_ref[...] = lax.dot_general(qn, w_ref[...], (((1,), (1,)), ((), ())),
                                 preferred_element_type=jnp.float32) + b_ref[...]


def _commit_proj(raw, xp, w, b):
    return pl.pallas_call(
        _commit_proj_body,
        out_shape=[jax.ShapeDtypeStruct((T, C), jnp.float32),
                   jax.ShapeDtypeStruct((1, 1), jnp.float32)],
    )(raw, xp, w, b.reshape(1, C))


def _l1_body(t_ref, u_ref, o_ref):
    j = pl.program_id(0)
    d = t_ref[...] - u_ref[...]
    ad = jnp.abs(d)
    l1 = jnp.where(ad < 1.0, 0.5 * d * d, ad - 0.5)
    s = jnp.sum(l1)

    @pl.when(j == 0)
    def _():
        o_ref[...] = s[None, None]

    @pl.when(j > 0)
    def _():
        o_ref[...] += s[None, None]


def _l1_sum(tgt, u):
    return pl.pallas_call(
        _l1_body,
        grid=(2,),
        in_specs=[pl.BlockSpec((T // 2, C), lambda j: (j, 0))] * 2,
        out_specs=pl.BlockSpec((1, 1), lambda j: (0, 0)),
        out_shape=jax.ShapeDtypeStruct((1, 1), jnp.float32),
    )(tgt, u)


def kernel(units, enc_in_W, enc_in_b, enc_rs_W, enc_rs_b, enc_rs_last_W,
           enc_rs_last_b, dec_in_W, dec_in_b, dec_rs_W, dec_rs_b,
           dec_rs_last_W, dec_rs_last_b, vq_in_W, vq_in_b, vq_embed,
           vq_out_W, vq_out_b):
    u = units[0]

    bf = jnp.bfloat16
    enc_Wt = jnp.transpose(enc_in_W.astype(bf), (0, 3, 1, 2))
    dec_Wt = jnp.transpose(dec_in_W.astype(bf), (0, 3, 1, 2))
    enc_in_b = enc_in_b.reshape(NL, 1, C2)
    dec_in_b = dec_in_b.reshape(NL, 1, C2)
    enc_rs_b = enc_rs_b.reshape(NL - 1, 1, C2)
    dec_rs_b = dec_rs_b.reshape(NL - 1, 1, C2)
    enc_rs = enc_rs_W[..., 0].astype(bf)
    dec_rs = dec_rs_W[..., 0].astype(bf)
    enc_last = enc_rs_last_W[..., 0].astype(bf)
    dec_last = dec_rs_last_W[..., 0].astype(bf)

    x_enc = _wn_stack(u, enc_Wt, enc_in_b, enc_rs, enc_rs_b, enc_last,
                      enc_rs_last_b.reshape(1, C))

    idx, xp = _vq_argmax(x_enc, vq_in_W, vq_in_b, vq_embed)
    raw = _sc_gather_rows(vq_embed, idx.reshape(T))
    q, commit_sum = _commit_proj(raw, xp, vq_out_W, vq_out_b)
    commit_loss = (commit_sum[0, 0] / (T * CODE)).astype(jnp.float32)

    tgt = _wn_stack(q, dec_Wt, dec_in_b, dec_rs, dec_rs_b, dec_last,
                    dec_rs_last_b.reshape(1, C))
    l1s = _l1_sum(tgt, u)
    l1_loss = (l1s[0, 0] / T).astype(jnp.float32)

    return (l1_loss, commit_loss)


# speedup vs baseline: 1.0340x; 1.0340x over previous
import functools
import jax
import jax.numpy as jnp
from jax import lax
from jax.experimental import pallas as pl
from jax.experimental.pallas import tpu as pltpu
from jax.experimental.pallas import tpu_sc as plsc

T = 2048
C = 768
C2 = 2 * C
K = 5
NL = 3
TPAD = T + 8
NT = 4
TB = T // NT
CODE = 32
NCODES = 8192
KB = 2048
NKB = NCODES // KB


def _gate(xin):
    return jnp.tanh(xin[:, :C]) * (
        0.5 * (1.0 + jnp.tanh(xin[:, C:] * 0.5)))


def _wn_body(x0_ref, w_ref, b_ref, rsw_ref, rsb_ref, rslw_ref, rslb_ref,
             o_ref, xpad_s, xin_s, acc_s, xres_s):
        l = pl.program_id(0)
        k = pl.program_id(1)

        @pl.when((l == 0) & (k == 0))
        def _():
            xpad_s[...] = jnp.zeros((TPAD, C), jnp.bfloat16)
            xpad_s[2:2 + T, :] = x0_ref[...].astype(jnp.bfloat16)
            xres_s[...] = x0_ref[...]
            acc_s[...] = jnp.zeros((T, C), jnp.float32)

        w = w_ref[0, 0]
        for i in range(K):
            @pl.when(k == i)
            def _(i=i):
                contrib = lax.dot_general(
                    xpad_s[i:i + T, :], w, (((1,), (1,)), ((), ())),
                    preferred_element_type=jnp.float32)
                if i == 0:
                    xin_s[...] = b_ref[0] + contrib
                else:
                    xin_s[...] += contrib

        @pl.when(k == K - 1)
        def _():
            @pl.when(l < NL - 1)
            def _():
                rsw = rsw_ref[0]
                rsb = rsb_ref[0]
                for ci in range(NT):
                    r0 = ci * TB
                    a = _gate(xin_s[r0:r0 + TB, :]).astype(jnp.bfloat16)
                    rsa = lax.dot_general(
                        a, rsw, (((1,), (1,)), ((), ())),
                        preferred_element_type=jnp.float32) + rsb
                    xn = xres_s[r0:r0 + TB, :] + rsa[:, :C]
                    xres_s[r0:r0 + TB, :] = xn
                    xpad_s[2 + r0:2 + r0 + TB, :] = xn.astype(jnp.bfloat16)
                    acc_s[r0:r0 + TB, :] += rsa[:, C:]

            @pl.when(l == NL - 1)
            def _():
                rslw = rslw_ref[...]
                rslb = rslb_ref[...]
                for ci in range(NT):
                    r0 = ci * TB
                    a = _gate(xin_s[r0:r0 + TB, :]).astype(jnp.bfloat16)
                    rsa = lax.dot_general(
                        a, rslw, (((1,), (1,)), ((), ())),
                        preferred_element_type=jnp.float32) + rslb
                    out = acc_s[r0:r0 + TB, :] + rsa
                    o_ref[r0:r0 + TB, :] = out


def _wn_stack(x0, W_all, b_all, rs_W, rs_b, rslw, rslb):
    in_specs = [
        pl.BlockSpec((T, C), lambda l, k: (0, 0)),
        pl.BlockSpec((1, 1, C2, C), lambda l, k: (l, k, 0, 0)),
        pl.BlockSpec((1, 1, C2), lambda l, k: (l, 0, 0)),
        pl.BlockSpec((1, C2, C), lambda l, k: (jnp.minimum(l, NL - 2), 0, 0)),
        pl.BlockSpec((1, 1, C2), lambda l, k: (jnp.minimum(l, NL - 2), 0, 0)),
        pl.BlockSpec((C, C), lambda l, k: (0, 0)),
        pl.BlockSpec((1, C), lambda l, k: (0, 0)),
    ]
    args = [x0, W_all, b_all, rs_W, rs_b, rslw, rslb]
    out_spec = pl.BlockSpec((T, C), lambda l, k: (0, 0))
    out_shape = jax.ShapeDtypeStruct((T, C), jnp.float32)
    return pl.pallas_call(
        _wn_body,
        grid=(NL, K),
        in_specs=in_specs,
        out_specs=out_spec,
        out_shape=out_shape,
        scratch_shapes=[
            pltpu.VMEM((TPAD, C), jnp.bfloat16),
            pltpu.VMEM((T, C2), jnp.float32),
            pltpu.VMEM((T, C), jnp.float32),
            pltpu.VMEM((T, C), jnp.float32),
        ],
        compiler_params=pltpu.CompilerParams(
            dimension_semantics=("arbitrary", "arbitrary")),
    )(*args)


def _dist_body(x_ref, w_ref, b_ref, e_ref, idx_ref, xp_ref, bv_ref, xn_s):
    j = pl.program_id(0)

    @pl.when(j == 0)
    def _():
        xp = lax.dot_general(x_ref[...], w_ref[...], (((1,), (1,)), ((), ())),
                             preferred_element_type=jnp.float32) + b_ref[...]
        nn = jnp.sqrt(jnp.sum(xp * xp, axis=1, keepdims=True))
        xp_ref[...] = xp
        xn_s[...] = xp / jnp.clip(nn, 1e-12, None)

    e = e_ref[...]
    n = jnp.sqrt(jnp.sum(e * e, axis=1, keepdims=True))
    en = e / jnp.clip(n, 1e-12, None)
    d = lax.dot_general(xn_s[...], en, (((1,), (1,)), ((), ())),
                        preferred_element_type=jnp.float32)
    bv = jnp.max(d, axis=1, keepdims=True)
    bi = jnp.argmax(d, axis=1).astype(jnp.int32)[:, None] + j * KB

    @pl.when(j == 0)
    def _():
        bv_ref[...] = bv
        idx_ref[...] = bi

    @pl.when(j > 0)
    def _():
        better = bv > bv_ref[...]
        idx_ref[...] = jnp.where(better, bi, idx_ref[...])
        bv_ref[...] = jnp.where(better, bv, bv_ref[...])


def _vq_argmax(x_enc, vqw, vqb, embed):
    return pl.pallas_call(
        _dist_body,
        grid=(NKB,),
        in_specs=[
            pl.BlockSpec((T, C), lambda j: (0, 0)),
            pl.BlockSpec((CODE, C), lambda j: (0, 0)),
            pl.BlockSpec((1, CODE), lambda j: (0, 0)),
            pl.BlockSpec((KB, CODE), lambda j: (j, 0)),
        ],
        out_specs=[pl.BlockSpec((T, 1), lambda j: (0, 0)),
                   pl.BlockSpec((T, CODE), lambda j: (0, 0))],
        out_shape=[jax.ShapeDtypeStruct((T, 1), jnp.int32),
                   jax.ShapeDtypeStruct((T, CODE), jnp.float32)],
        scratch_shapes=[pltpu.VMEM((T, 1), jnp.float32),
                        pltpu.VMEM((T, CODE), jnp.float32)],
    )(x_enc, vqw, vqb.reshape(1, CODE), embed)


_NC = 2
_NS = 16
_NW = _NC * _NS
_BPW = T // _NW


def _sc_gather_rows(table, idx):
    mesh = plsc.VectorSubcoreMesh(core_axis_name="c", subcore_axis_name="s")

    @functools.partial(
        pl.kernel, mesh=mesh,
        compiler_params=pltpu.CompilerParams(use_tc_tiling_on_sc=False),
        out_type=jax.ShapeDtypeStruct((T, CODE), jnp.float32),
        scratch_types=[
            pltpu.VMEM((_BPW,), jnp.int32),
            pltpu.VMEM((_BPW, CODE), jnp.float32),
            pltpu.SemaphoreType.DMA,
        ],
    )
    def gather_k(table_hbm, idx_hbm, out_hbm, idx_v, rows_v, sem):
        wid = lax.axis_index("s") * _NC + lax.axis_index("c")
        base = wid * _BPW
        pltpu.sync_copy(idx_hbm.at[pl.ds(base, _BPW)], idx_v)
        pltpu.async_copy(table_hbm.at[idx_v], rows_v, sem).wait()
        pltpu.sync_copy(rows_v, out_hbm.at[pl.ds(base, _BPW)])

    return gather_k(table, idx)


def _commit_proj_body(raw_ref, xp_ref, w_ref, b_ref, q_ref, cm_ref):
    raw = raw_ref[...]
    n = jnp.sqrt(jnp.sum(raw * raw, axis=1, keepdims=True))
    qn = raw / jnp.clip(n, 1e-12, None)
    diff = qn - xp_ref[...]
    cm_ref[...] = jnp.sum(diff * diff)[None, None]
    q_ref[...] = lax.dot_general(qn, w_ref[...], (((1,), (1,)), ((), ())),
                                 preferred_element_type=jnp.float32) + b_ref[...]


def _commit_proj(raw, xp, w, b):
    return pl.pallas_call(
        _commit_proj_body,
        out_shape=[jax.ShapeDtypeStruct((T, C), jnp.float32),
                   jax.ShapeDtypeStruct((1, 1), jnp.float32)],
    )(raw, xp, w, b.reshape(1, C))


def _l1_body(t_ref, u_ref, o_ref):
    j = pl.program_id(0)
    d = t_ref[...] - u_ref[...]
    ad = jnp.abs(d)
    l1 = jnp.where(ad < 1.0, 0.5 * d * d, ad - 0.5)
    s = jnp.sum(l1)

    @pl.when(j == 0)
    def _():
        o_ref[...] = s[None, None]

    @pl.when(j > 0)
    def _():
        o_ref[...] += s[None, None]


def _l1_sum(tgt, u):
    return pl.pallas_call(
        _l1_body,
        grid=(2,),
        in_specs=[pl.BlockSpec((T // 2, C), lambda j: (j, 0))] * 2,
        out_specs=pl.BlockSpec((1, 1), lambda j: (0, 0)),
        out_shape=jax.ShapeDtypeStruct((1, 1), jnp.float32),
    )(tgt, u)


def kernel(units, enc_in_W, enc_in_b, enc_rs_W, enc_rs_b, enc_rs_last_W,
           enc_rs_last_b, dec_in_W, dec_in_b, dec_rs_W, dec_rs_b,
           dec_rs_last_W, dec_rs_last_b, vq_in_W, vq_in_b, vq_embed,
           vq_out_W, vq_out_b):
    u = units[0]

    bf = jnp.bfloat16
    enc_Wt = jnp.transpose(enc_in_W.astype(bf), (0, 3, 1, 2))
    dec_Wt = jnp.transpose(dec_in_W.astype(bf), (0, 3, 1, 2))
    enc_in_b = enc_in_b.reshape(NL, 1, C2)
    dec_in_b = dec_in_b.reshape(NL, 1, C2)
    enc_rs_b = enc_rs_b.reshape(NL - 1, 1, C2)
    dec_rs_b = dec_rs_b.reshape(NL - 1, 1, C2)
    enc_rs = enc_rs_W[..., 0].astype(bf)
    dec_rs = dec_rs_W[..., 0].astype(bf)
    enc_last = enc_rs_last_W[..., 0].astype(bf)
    dec_last = dec_rs_last_W[..., 0].astype(bf)

    x_enc = _wn_stack(u, enc_Wt, enc_in_b, enc_rs, enc_rs_b, enc_last,
                      enc_rs_last_b.reshape(1, C))

    idx, xp = _vq_argmax(x_enc, vq_in_W, vq_in_b, vq_embed)
    raw = _sc_gather_rows(vq_embed, idx.reshape(T))
    q, commit_sum = _commit_proj(raw, xp, vq_out_W, vq_out_b)
    commit_loss = (commit_sum[0, 0] / (T * CODE)).astype(jnp.float32)

    tgt = _wn_stack(q, dec_Wt, dec_in_b, dec_rs, dec_rs_b, dec_last,
                    dec_rs_last_b.reshape(1, C))
    l1s = _l1_sum(tgt, u)
    l1_loss = (l1s[0, 0] / T).astype(jnp.float32)

    return (l1_loss, commit_loss)
